# Initial kernel scaffold; baseline (speedup 1.0000x reference)
import jax
import jax.numpy as jnp
from jax.experimental import pallas as pl


def kernel(z_t, z_tp1, embed_w, W1, b1, W2, b2, codebook):
    raise NotImplementedError("write your pallas kernel here")


# speedup vs baseline: 39.8345x; 39.8345x over previous
import functools

import jax
import jax.numpy as jnp
from jax import lax
from jax.experimental import pallas as pl
from jax.experimental.pallas import tpu as pltpu
from jax.experimental.pallas import tpu_sc as plsc

_K = 8192
_A = 1024
_DIM = 256
_J = 1024

_NC, _NS, _L = 2, 16, 16
_NW = _NC * _NS
_ROWS = 2048
_RPW = _ROWS // _NW

@functools.cache
def _get_sc_counts():
    mesh = plsc.VectorSubcoreMesh(core_axis_name="c", subcore_axis_name="s",
                                  num_cores=_NC, num_subcores=_NS)

    @functools.partial(
        pl.kernel,
        out_type=jax.ShapeDtypeStruct((_ROWS, _K), jnp.float32),
        mesh=mesh,
        scratch_types=[
            pltpu.VMEM((_J,), jnp.int32),
            pltpu.VMEM((_K,), jnp.float32),
        ],
        compiler_params=pltpu.CompilerParams(needs_layout_passes=False),
    )
    def _sc_counts(z_hbm, out_hbm, idx_v, cnt_v):
        _sc_counts_body(z_hbm, out_hbm, idx_v, cnt_v)

    return _sc_counts


def _sc_counts_body(z_hbm, out_hbm, idx_v, cnt_v):
    wid = lax.axis_index("s") * _NC + lax.axis_index("c")
    zero16 = jnp.zeros((_L,), jnp.float32)
    one16 = jnp.ones((_L,), jnp.float32)
    mone16 = -one16

    def _zero(i, c):
        cnt_v[pl.ds(i * _L, _L)] = zero16
        return c

    lax.fori_loop(0, _K // _L, _zero, 0)

    def _row(r, c):
        row = wid * _RPW + r
        pltpu.sync_copy(z_hbm.at[row], idx_v)

        def _scat(j, cc):
            ii = idx_v[pl.ds(j * _L, _L)]
            plsc.addupdate_scatter(cnt_v, [ii], one16)
            return cc

        lax.fori_loop(0, _J // _L, _scat, 0)
        pltpu.sync_copy(cnt_v, out_hbm.at[row])

        def _unscat(j, cc):
            ii = idx_v[pl.ds(j * _L, _L)]
            plsc.addupdate_scatter(cnt_v, [ii], mone16)
            return cc

        lax.fori_loop(0, _J // _L, _unscat, 0)
        return c

    lax.fori_loop(0, _RPW, _row, 0)


_BB = 128


def _tc_body(ct_ref, ctp_ref, emb_ref, w1t_ref, b1_ref, w2t_ref, b2_ref,
             cbt_ref, cb_ref, idx_ref, q_ref):
    hp = lax.Precision.HIGHEST
    dp = lax.Precision.DEFAULT
    inv = 1.0 / float(_J)
    et = jnp.dot(ct_ref[...], emb_ref[...], precision=hp) * inv
    etp = jnp.dot(ctp_ref[...], emb_ref[...], precision=hp) * inv
    cc = jnp.concatenate([et, etp], axis=1)
    h = jnp.dot(cc, w1t_ref[...], precision=dp) + b1_ref[...]
    h = 0.5 * h * (1.0 + lax.erf(h * (2.0 ** -0.5)))
    e = jnp.dot(h, w2t_ref[...], precision=dp) + b2_ref[...]
    ecb = jnp.dot(e, cbt_ref[...], precision=dp)
    e2 = jnp.sum(e * e, axis=1, keepdims=True)
    cbt = cbt_ref[...]
    cb2 = jnp.sum(cbt * cbt, axis=0, keepdims=True)
    d2 = e2 - 2.0 * ecb + cb2
    m = jnp.min(d2, axis=1, keepdims=True)
    kio = lax.broadcasted_iota(jnp.int32, (_BB, _A), 1)
    cand = jnp.where(d2 <= m, kio, _A)
    idx2 = jnp.min(cand, axis=1, keepdims=True)
    idx_ref[...] = idx2
    onehot = (kio == idx2).astype(jnp.float32)
    q_ref[...] = jnp.dot(onehot, cb_ref[...], precision=hp)


def _tc_call(counts, emb, w1t, b1, w2t, b2, cbt, cb):
    nblk = _A // _BB
    return pl.pallas_call(
        _tc_body,
        grid=(nblk,),
        in_specs=[
            pl.BlockSpec((_BB, _K), lambda i: (i, 0)),
            pl.BlockSpec((_BB, _K), lambda i, n=nblk: (i + n, 0)),
            pl.BlockSpec((_K, _DIM), lambda i: (0, 0)),
            pl.BlockSpec((2 * _DIM, _DIM), lambda i: (0, 0)),
            pl.BlockSpec((1, _DIM), lambda i: (0, 0)),
            pl.BlockSpec((_DIM, _DIM), lambda i: (0, 0)),
            pl.BlockSpec((1, _DIM), lambda i: (0, 0)),
            pl.BlockSpec((_DIM, _A), lambda i: (0, 0)),
            pl.BlockSpec((_A, _DIM), lambda i: (0, 0)),
        ],
        out_specs=[
            pl.BlockSpec((_BB, 1), lambda i: (i, 0)),
            pl.BlockSpec((_BB, _DIM), lambda i: (i, 0)),
        ],
        out_shape=[
            jax.ShapeDtypeStruct((_A, 1), jnp.int32),
            jax.ShapeDtypeStruct((_A, _DIM), jnp.float32),
        ],
    )(counts, counts, emb, w1t, b1, w2t, b2, cbt, cb)


def kernel(z_t, z_tp1, embed_w, W1, b1, W2, b2, codebook):
    B = z_t.shape[0]
    z_all = jnp.concatenate(
        [z_t.reshape(B, -1).astype(jnp.int32),
         z_tp1.reshape(B, -1).astype(jnp.int32)], axis=0)
    counts = _get_sc_counts()(z_all)
    idx2, q = _tc_call(counts, embed_w, W1.T, b1.reshape(1, -1), W2.T,
                       b2.reshape(1, -1), codebook.T, codebook)
    return (idx2.reshape(B), q)


# speedup vs baseline: 53.2322x; 1.3363x over previous
import functools

import jax
import jax.numpy as jnp
from jax import lax
from jax.experimental import pallas as pl
from jax.experimental.pallas import tpu as pltpu
from jax.experimental.pallas import tpu_sc as plsc

_K = 8192
_A = 1024
_DIM = 256
_J = 1024

_NC, _NS, _L = 2, 16, 16
_NW = _NC * _NS
_ROWS = 2048
_RPW = _ROWS // _NW

@functools.cache
def _get_sc_counts():
    mesh = plsc.VectorSubcoreMesh(core_axis_name="c", subcore_axis_name="s",
                                  num_cores=_NC, num_subcores=_NS)

    nbuf = 4

    @functools.partial(
        pl.kernel,
        out_type=jax.ShapeDtypeStruct((_ROWS, _K), jnp.float32),
        mesh=mesh,
        scratch_types=[
            pltpu.VMEM((_RPW, _J), jnp.int32),
            [pltpu.VMEM((_K,), jnp.float32) for _ in range(nbuf)],
            [pltpu.SemaphoreType.DMA for _ in range(nbuf)],
        ],
        compiler_params=pltpu.CompilerParams(needs_layout_passes=False),
    )
    def _sc_counts(z_hbm, out_hbm, idx_all, cnts, sems):
        wid = lax.axis_index("s") * _NC + lax.axis_index("c")
        base = wid * _RPW
        zero16 = jnp.zeros((_L,), jnp.float32)
        one16 = jnp.ones((_L,), jnp.float32)
        mone16 = -one16

        pltpu.sync_copy(z_hbm.at[pl.ds(base, _RPW)], idx_all)

        def _zero(i, c):
            for b in range(nbuf):
                cnts[b][pl.ds(i * _L, _L)] = zero16
            return c

        lax.fori_loop(0, _K // _L, _zero, 0)

        def _apply(rl, cref, val16):
            def body(j, c):
                ii = idx_all[rl, pl.ds(j * _L, _L)]
                plsc.addupdate_scatter(cref, [ii], val16)
                return c

            lax.fori_loop(0, _J // _L, body, 0)

        def _group(g, c):
            for b in range(nbuf):
                rl = g * nbuf + b

                @pl.when(g > 0)
                def _():
                    pltpu.make_async_copy(
                        cnts[b], out_hbm.at[base + rl - nbuf], sems[b]).wait()
                    _apply(rl - nbuf, cnts[b], mone16)

                _apply(rl, cnts[b], one16)
                pltpu.async_copy(cnts[b], out_hbm.at[base + rl], sems[b])
            return c

        lax.fori_loop(0, _RPW // nbuf, _group, 0)
        for b in range(nbuf):
            pltpu.make_async_copy(
                cnts[b], out_hbm.at[base + _RPW - nbuf + b], sems[b]).wait()

    return _sc_counts


_BB = 128


def _tc_body(ct_ref, ctp_ref, emb_ref, w1t_ref, b1_ref, w2t_ref, b2_ref,
             cbt_ref, cb_ref, idx_ref, q_ref):
    hp = lax.Precision.HIGHEST
    dp = lax.Precision.DEFAULT
    inv = 1.0 / float(_J)
    et = jnp.dot(ct_ref[...], emb_ref[...], precision=hp) * inv
    etp = jnp.dot(ctp_ref[...], emb_ref[...], precision=hp) * inv
    cc = jnp.concatenate([et, etp], axis=1)
    h = jnp.dot(cc, w1t_ref[...], precision=dp) + b1_ref[...]
    h = 0.5 * h * (1.0 + lax.erf(h * (2.0 ** -0.5)))
    e = jnp.dot(h, w2t_ref[...], precision=dp) + b2_ref[...]
    ecb = jnp.dot(e, cbt_ref[...], precision=dp)
    e2 = jnp.sum(e * e, axis=1, keepdims=True)
    cbt = cbt_ref[...]
    cb2 = jnp.sum(cbt * cbt, axis=0, keepdims=True)
    d2 = e2 - 2.0 * ecb + cb2
    m = jnp.min(d2, axis=1, keepdims=True)
    kio = lax.broadcasted_iota(jnp.int32, (_BB, _A), 1)
    cand = jnp.where(d2 <= m, kio, _A)
    idx2 = jnp.min(cand, axis=1, keepdims=True)
    idx_ref[...] = idx2
    onehot = (kio == idx2).astype(jnp.float32)
    q_ref[...] = jnp.dot(onehot, cb_ref[...], precision=hp)


def _tc_call(counts, emb, w1t, b1, w2t, b2, cbt, cb):
    nblk = _A // _BB
    return pl.pallas_call(
        _tc_body,
        grid=(nblk,),
        in_specs=[
            pl.BlockSpec((_BB, _K), lambda i: (i, 0)),
            pl.BlockSpec((_BB, _K), lambda i, n=nblk: (i + n, 0)),
            pl.BlockSpec((_K, _DIM), lambda i: (0, 0)),
            pl.BlockSpec((2 * _DIM, _DIM), lambda i: (0, 0)),
            pl.BlockSpec((1, _DIM), lambda i: (0, 0)),
            pl.BlockSpec((_DIM, _DIM), lambda i: (0, 0)),
            pl.BlockSpec((1, _DIM), lambda i: (0, 0)),
            pl.BlockSpec((_DIM, _A), lambda i: (0, 0)),
            pl.BlockSpec((_A, _DIM), lambda i: (0, 0)),
        ],
        out_specs=[
            pl.BlockSpec((_BB, 1), lambda i: (i, 0)),
            pl.BlockSpec((_BB, _DIM), lambda i: (i, 0)),
        ],
        out_shape=[
            jax.ShapeDtypeStruct((_A, 1), jnp.int32),
            jax.ShapeDtypeStruct((_A, _DIM), jnp.float32),
        ],
    )(counts, counts, emb, w1t, b1, w2t, b2, cbt, cb)


def kernel(z_t, z_tp1, embed_w, W1, b1, W2, b2, codebook):
    B = z_t.shape[0]
    z_all = jnp.concatenate(
        [z_t.reshape(B, -1).astype(jnp.int32),
         z_tp1.reshape(B, -1).astype(jnp.int32)], axis=0)
    counts = _get_sc_counts()(z_all)
    idx2, q = _tc_call(counts, embed_w, W1.T, b1.reshape(1, -1), W2.T,
                       b2.reshape(1, -1), codebook.T, codebook)
    return (idx2.reshape(B), q)


# speedup vs baseline: 69.5377x; 1.3063x over previous
import functools

import jax
import jax.numpy as jnp
from jax import lax
from jax.experimental import pallas as pl
from jax.experimental.pallas import tpu as pltpu
from jax.experimental.pallas import tpu_sc as plsc

_K = 8192
_A = 1024
_DIM = 256
_J = 1024

_NC, _NS, _L = 2, 16, 16
_NW = _NC * _NS
_ROWS = 2048
_RPW = _ROWS // _NW

@functools.cache
def _get_sc_counts():
    mesh = plsc.VectorSubcoreMesh(core_axis_name="c", subcore_axis_name="s",
                                  num_cores=_NC, num_subcores=_NS)

    nbuf = 4

    @functools.partial(
        pl.kernel,
        out_type=jax.ShapeDtypeStruct((_ROWS, _K), jnp.float32),
        mesh=mesh,
        scratch_types=[
            pltpu.VMEM((_RPW, _J), jnp.int32),
            [pltpu.VMEM((_K,), jnp.float32) for _ in range(nbuf)],
            [pltpu.SemaphoreType.DMA for _ in range(nbuf)],
        ],
        compiler_params=pltpu.CompilerParams(needs_layout_passes=False),
    )
    def _sc_counts(z_hbm, out_hbm, idx_all, cnts, sems):
        wid = lax.axis_index("s") * _NC + lax.axis_index("c")
        base = wid * _RPW
        zero16 = jnp.zeros((_L,), jnp.float32)
        one16 = jnp.ones((_L,), jnp.float32)
        mone16 = -one16

        pltpu.sync_copy(z_hbm.at[pl.ds(base, _RPW)], idx_all)

        def _zero(i, c):
            for b in range(nbuf):
                cnts[b][pl.ds(i * _L, _L)] = zero16
            return c

        lax.fori_loop(0, _K // _L, _zero, 0)

        unroll = 8

        def _apply(rl, cref, val16):
            def body(j, c):
                for u in range(unroll):
                    ii = idx_all[rl, pl.ds((j * unroll + u) * _L, _L)]
                    plsc.addupdate_scatter(cref, [ii], val16)
                return c

            lax.fori_loop(0, _J // _L // unroll, body, 0)

        def _group(g, c):
            for b in range(nbuf):
                rl = g * nbuf + b

                @pl.when(g > 0)
                def _():
                    pltpu.make_async_copy(
                        cnts[b], out_hbm.at[base + rl - nbuf], sems[b]).wait()
                    _apply(rl - nbuf, cnts[b], mone16)

                _apply(rl, cnts[b], one16)
                pltpu.async_copy(cnts[b], out_hbm.at[base + rl], sems[b])
            return c

        lax.fori_loop(0, _RPW // nbuf, _group, 0)
        for b in range(nbuf):
            pltpu.make_async_copy(
                cnts[b], out_hbm.at[base + _RPW - nbuf + b], sems[b]).wait()

    return _sc_counts


_BB = 128


def _tc_body(ct_ref, ctp_ref, emb_ref, w1t_ref, b1_ref, w2t_ref, b2_ref,
             cbt_ref, cb_ref, idx_ref, q_ref):
    hp = lax.Precision.HIGHEST
    dp = lax.Precision.DEFAULT
    inv = 1.0 / float(_J)
    emb = emb_ref[...]
    ehi = emb.astype(jnp.bfloat16)
    elo = (emb - ehi.astype(jnp.float32)).astype(jnp.bfloat16)
    ct16 = ct_ref[...].astype(jnp.bfloat16)
    ctp16 = ctp_ref[...].astype(jnp.bfloat16)
    f32 = jnp.float32
    et = (jnp.dot(ct16, ehi, preferred_element_type=f32)
          + jnp.dot(ct16, elo, preferred_element_type=f32)) * inv
    etp = (jnp.dot(ctp16, ehi, preferred_element_type=f32)
           + jnp.dot(ctp16, elo, preferred_element_type=f32)) * inv
    cc = jnp.concatenate([et, etp], axis=1)
    h = jnp.dot(cc, w1t_ref[...], precision=dp) + b1_ref[...]
    h = 0.5 * h * (1.0 + lax.erf(h * (2.0 ** -0.5)))
    e = jnp.dot(h, w2t_ref[...], precision=dp) + b2_ref[...]
    ecb = jnp.dot(e, cbt_ref[...], precision=dp)
    e2 = jnp.sum(e * e, axis=1, keepdims=True)
    cbt = cbt_ref[...]
    cb2 = jnp.sum(cbt * cbt, axis=0, keepdims=True)
    d2 = e2 - 2.0 * ecb + cb2
    m = jnp.min(d2, axis=1, keepdims=True)
    kio = lax.broadcasted_iota(jnp.int32, (_BB, _A), 1)
    cand = jnp.where(d2 <= m, kio, _A)
    idx2 = jnp.min(cand, axis=1, keepdims=True)
    idx_ref[...] = idx2
    onehot = (kio == idx2).astype(jnp.float32)
    q_ref[...] = jnp.dot(onehot, cb_ref[...], precision=hp)


def _tc_call(counts, emb, w1t, b1, w2t, b2, cbt, cb):
    nblk = _A // _BB
    return pl.pallas_call(
        _tc_body,
        grid=(nblk,),
        in_specs=[
            pl.BlockSpec((_BB, _K), lambda i: (i, 0)),
            pl.BlockSpec((_BB, _K), lambda i, n=nblk: (i + n, 0)),
            pl.BlockSpec((_K, _DIM), lambda i: (0, 0)),
            pl.BlockSpec((2 * _DIM, _DIM), lambda i: (0, 0)),
            pl.BlockSpec((1, _DIM), lambda i: (0, 0)),
            pl.BlockSpec((_DIM, _DIM), lambda i: (0, 0)),
            pl.BlockSpec((1, _DIM), lambda i: (0, 0)),
            pl.BlockSpec((_DIM, _A), lambda i: (0, 0)),
            pl.BlockSpec((_A, _DIM), lambda i: (0, 0)),
        ],
        out_specs=[
            pl.BlockSpec((_BB, 1), lambda i: (i, 0)),
            pl.BlockSpec((_BB, _DIM), lambda i: (i, 0)),
        ],
        out_shape=[
            jax.ShapeDtypeStruct((_A, 1), jnp.int32),
            jax.ShapeDtypeStruct((_A, _DIM), jnp.float32),
        ],
    )(counts, counts, emb, w1t, b1, w2t, b2, cbt, cb)


def kernel(z_t, z_tp1, embed_w, W1, b1, W2, b2, codebook):
    B = z_t.shape[0]
    z_all = jnp.concatenate(
        [z_t.reshape(B, -1).astype(jnp.int32),
         z_tp1.reshape(B, -1).astype(jnp.int32)], axis=0)
    counts = _get_sc_counts()(z_all)
    idx2, q = _tc_call(counts, embed_w, W1.T, b1.reshape(1, -1), W2.T,
                       b2.reshape(1, -1), codebook.T, codebook)
    return (idx2.reshape(B), q)


# speedup vs baseline: 71.3099x; 1.0255x over previous
import functools

import jax
import jax.numpy as jnp
from jax import lax
from jax.experimental import pallas as pl
from jax.experimental.pallas import tpu as pltpu
from jax.experimental.pallas import tpu_sc as plsc

_K = 8192
_A = 1024
_DIM = 256
_J = 1024

_NC, _NS, _L = 2, 16, 16
_NW = _NC * _NS
_ROWS = 1024
_RPW = _ROWS // _NW

@functools.cache
def _get_sc_counts():
    mesh = plsc.VectorSubcoreMesh(core_axis_name="c", subcore_axis_name="s",
                                  num_cores=_NC, num_subcores=_NS)

    nbuf = 4

    @functools.partial(
        pl.kernel,
        out_type=jax.ShapeDtypeStruct((_ROWS, _K), jnp.float32),
        mesh=mesh,
        scratch_types=[
            pltpu.VMEM((_RPW, _J), jnp.int32),
            [pltpu.VMEM((_K,), jnp.float32) for _ in range(nbuf)],
            [pltpu.SemaphoreType.DMA for _ in range(nbuf)],
        ],
        compiler_params=pltpu.CompilerParams(needs_layout_passes=False),
    )
    def _sc_counts(z_hbm, out_hbm, idx_all, cnts, sems):
        wid = lax.axis_index("s") * _NC + lax.axis_index("c")
        base = wid * _RPW
        zero16 = jnp.zeros((_L,), jnp.float32)
        one16 = jnp.ones((_L,), jnp.float32)
        mone16 = -one16

        pltpu.sync_copy(z_hbm.at[pl.ds(base, _RPW)], idx_all)

        def _zero(i, c):
            for b in range(nbuf):
                cnts[b][pl.ds(i * _L, _L)] = zero16
            return c

        lax.fori_loop(0, _K // _L, _zero, 0)

        unroll = 8

        def _apply(rl, cref, val16):
            def body(j, c):
                for u in range(unroll):
                    ii = idx_all[rl, pl.ds((j * unroll + u) * _L, _L)]
                    plsc.addupdate_scatter(cref, [ii], val16)
                return c

            lax.fori_loop(0, _J // _L // unroll, body, 0)

        def _group(g, c):
            for b in range(nbuf):
                rl = g * nbuf + b

                @pl.when(g > 0)
                def _():
                    pltpu.make_async_copy(
                        cnts[b], out_hbm.at[base + rl - nbuf], sems[b]).wait()
                    _apply(rl - nbuf, cnts[b], mone16)

                _apply(rl, cnts[b], one16)
                pltpu.async_copy(cnts[b], out_hbm.at[base + rl], sems[b])
            return c

        lax.fori_loop(0, _RPW // nbuf, _group, 0)
        for b in range(nbuf):
            pltpu.make_async_copy(
                cnts[b], out_hbm.at[base + _RPW - nbuf + b], sems[b]).wait()

    return _sc_counts


_BB = 128


def _tc_body(ct_ref, ctp_ref, emb_ref, w1t_ref, b1_ref, w2t_ref, b2_ref,
             cbt_ref, cb_ref, idx_ref, q_ref):
    hp = lax.Precision.HIGHEST
    dp = lax.Precision.DEFAULT
    inv = 1.0 / float(_J)
    emb = emb_ref[...]
    ehi = emb.astype(jnp.bfloat16)
    elo = (emb - ehi.astype(jnp.float32)).astype(jnp.bfloat16)
    ct16 = ct_ref[...].astype(jnp.bfloat16)
    ctp16 = ctp_ref[...].astype(jnp.bfloat16)
    f32 = jnp.float32
    et = (jnp.dot(ct16, ehi, preferred_element_type=f32)
          + jnp.dot(ct16, elo, preferred_element_type=f32)) * inv
    etp = (jnp.dot(ctp16, ehi, preferred_element_type=f32)
           + jnp.dot(ctp16, elo, preferred_element_type=f32)) * inv
    cc = jnp.concatenate([et, etp], axis=1)
    h = jnp.dot(cc, w1t_ref[...], precision=dp) + b1_ref[...]
    h = 0.5 * h * (1.0 + lax.erf(h * (2.0 ** -0.5)))
    e = jnp.dot(h, w2t_ref[...], precision=dp) + b2_ref[...]
    ecb = jnp.dot(e, cbt_ref[...], precision=dp)
    e2 = jnp.sum(e * e, axis=1, keepdims=True)
    cbt = cbt_ref[...]
    cb2 = jnp.sum(cbt * cbt, axis=0, keepdims=True)
    d2 = e2 - 2.0 * ecb + cb2
    m = jnp.min(d2, axis=1, keepdims=True)
    kio = lax.broadcasted_iota(jnp.int32, (_BB, _A), 1)
    cand = jnp.where(d2 <= m, kio, _A)
    idx2 = jnp.min(cand, axis=1, keepdims=True)
    idx_ref[...] = idx2
    onehot = (kio == idx2).astype(jnp.float32)
    q_ref[...] = jnp.dot(onehot, cb_ref[...], precision=hp)


def _tc_call(counts_t, counts_tp, emb, w1t, b1, w2t, b2, cbt, cb):
    nblk = _A // _BB
    return pl.pallas_call(
        _tc_body,
        grid=(nblk,),
        in_specs=[
            pl.BlockSpec((_BB, _K), lambda i: (i, 0)),
            pl.BlockSpec((_BB, _K), lambda i: (i, 0)),
            pl.BlockSpec((_K, _DIM), lambda i: (0, 0)),
            pl.BlockSpec((2 * _DIM, _DIM), lambda i: (0, 0)),
            pl.BlockSpec((1, _DIM), lambda i: (0, 0)),
            pl.BlockSpec((_DIM, _DIM), lambda i: (0, 0)),
            pl.BlockSpec((1, _DIM), lambda i: (0, 0)),
            pl.BlockSpec((_DIM, _A), lambda i: (0, 0)),
            pl.BlockSpec((_A, _DIM), lambda i: (0, 0)),
        ],
        out_specs=[
            pl.BlockSpec((_BB, 1), lambda i: (i, 0)),
            pl.BlockSpec((_BB, _DIM), lambda i: (i, 0)),
        ],
        out_shape=[
            jax.ShapeDtypeStruct((_A, 1), jnp.int32),
            jax.ShapeDtypeStruct((_A, _DIM), jnp.float32),
        ],
    )(counts_t, counts_tp, emb, w1t, b1, w2t, b2, cbt, cb)


def kernel(z_t, z_tp1, embed_w, W1, b1, W2, b2, codebook):
    B = z_t.shape[0]
    sc = _get_sc_counts()
    counts_t = sc(z_t.reshape(B, -1).astype(jnp.int32))
    counts_tp = sc(z_tp1.reshape(B, -1).astype(jnp.int32))
    idx2, q = _tc_call(counts_t, counts_tp, embed_w, W1.T, b1.reshape(1, -1),
                       W2.T, b2.reshape(1, -1), codebook.T, codebook)
    return (idx2.reshape(B), q)


# speedup vs baseline: 72.5487x; 1.0174x over previous
import functools

import jax
import jax.numpy as jnp
from jax import lax
from jax.experimental import pallas as pl
from jax.experimental.pallas import tpu as pltpu
from jax.experimental.pallas import tpu_sc as plsc

_K = 8192
_A = 1024
_DIM = 256
_J = 1024

_NC, _NS, _L = 2, 16, 16
_NW = _NC * _NS
_ROWS = 1024
_RPW = _ROWS // _NW

@functools.cache
def _get_sc_counts():
    mesh = plsc.VectorSubcoreMesh(core_axis_name="c", subcore_axis_name="s",
                                  num_cores=_NC, num_subcores=_NS)

    nbuf = 4

    @functools.partial(
        pl.kernel,
        out_type=[jax.ShapeDtypeStruct((_ROWS, _K), jnp.float32),
                  jax.ShapeDtypeStruct((_ROWS, _K), jnp.float32)],
        mesh=mesh,
        scratch_types=[
            pltpu.VMEM((2 * _RPW, _J), jnp.int32),
            [pltpu.VMEM((_K,), jnp.float32) for _ in range(nbuf)],
            [pltpu.SemaphoreType.DMA for _ in range(nbuf)],
        ],
        compiler_params=pltpu.CompilerParams(needs_layout_passes=False),
    )
    def _sc_counts(z1_hbm, z2_hbm, o1_hbm, o2_hbm, idx_all, cnts, sems):
        wid = lax.axis_index("s") * _NC + lax.axis_index("c")
        base = wid * _RPW
        zero16 = jnp.zeros((_L,), jnp.float32)
        one16 = jnp.ones((_L,), jnp.float32)
        mone16 = -one16

        pltpu.sync_copy(z1_hbm.at[pl.ds(base, _RPW)],
                        idx_all.at[pl.ds(0, _RPW)])
        pltpu.sync_copy(z2_hbm.at[pl.ds(base, _RPW)],
                        idx_all.at[pl.ds(_RPW, _RPW)])

        def _zero(i, c):
            for b in range(nbuf):
                cnts[b][pl.ds(i * _L, _L)] = zero16
            return c

        lax.fori_loop(0, _K // _L, _zero, 0)

        unroll = 8

        def _apply(rl, cref, val16):
            def body(j, c):
                for u in range(unroll):
                    ii = idx_all[rl, pl.ds((j * unroll + u) * _L, _L)]
                    plsc.addupdate_scatter(cref, [ii], val16)
                return c

            lax.fori_loop(0, _J // _L // unroll, body, 0)

        def _phase(out_hbm, roff, cleanup):
            def _group(g, c):
                for b in range(nbuf):
                    rl = g * nbuf + b

                    @pl.when(g > 0)
                    def _():
                        pltpu.make_async_copy(
                            cnts[b], out_hbm.at[base + rl - nbuf],
                            sems[b]).wait()
                        _apply(roff + rl - nbuf, cnts[b], mone16)

                    _apply(roff + rl, cnts[b], one16)
                    pltpu.async_copy(cnts[b], out_hbm.at[base + rl], sems[b])
                return c

            lax.fori_loop(0, _RPW // nbuf, _group, 0)
            for b in range(nbuf):
                rl = _RPW - nbuf + b
                pltpu.make_async_copy(
                    cnts[b], out_hbm.at[base + rl], sems[b]).wait()
                if cleanup:
                    _apply(roff + rl, cnts[b], mone16)

        _phase(o1_hbm, 0, True)
        _phase(o2_hbm, _RPW, False)

    return _sc_counts


_BB = 128


def _tc_body(ct_ref, ctp_ref, emb_ref, w1t_ref, b1_ref, w2t_ref, b2_ref,
             cbt_ref, cb_ref, idx_ref, q_ref):
    hp = lax.Precision.HIGHEST
    dp = lax.Precision.DEFAULT
    inv = 1.0 / float(_J)
    emb = emb_ref[...]
    ehi = emb.astype(jnp.bfloat16)
    elo = (emb - ehi.astype(jnp.float32)).astype(jnp.bfloat16)
    ct16 = ct_ref[...].astype(jnp.bfloat16)
    ctp16 = ctp_ref[...].astype(jnp.bfloat16)
    f32 = jnp.float32
    et = (jnp.dot(ct16, ehi, preferred_element_type=f32)
          + jnp.dot(ct16, elo, preferred_element_type=f32)) * inv
    etp = (jnp.dot(ctp16, ehi, preferred_element_type=f32)
           + jnp.dot(ctp16, elo, preferred_element_type=f32)) * inv
    cc = jnp.concatenate([et, etp], axis=1)
    h = jnp.dot(cc, w1t_ref[...], precision=dp) + b1_ref[...]
    h = 0.5 * h * (1.0 + lax.erf(h * (2.0 ** -0.5)))
    e = jnp.dot(h, w2t_ref[...], precision=dp) + b2_ref[...]
    ecb = jnp.dot(e, cbt_ref[...], precision=dp)
    e2 = jnp.sum(e * e, axis=1, keepdims=True)
    cbt = cbt_ref[...]
    cb2 = jnp.sum(cbt * cbt, axis=0, keepdims=True)
    d2 = e2 - 2.0 * ecb + cb2
    m = jnp.min(d2, axis=1, keepdims=True)
    kio = lax.broadcasted_iota(jnp.int32, (_BB, _A), 1)
    cand = jnp.where(d2 <= m, kio, _A)
    idx2 = jnp.min(cand, axis=1, keepdims=True)
    idx_ref[...] = idx2
    onehot = (kio == idx2).astype(jnp.float32)
    q_ref[...] = jnp.dot(onehot, cb_ref[...], precision=hp)


def _tc_call(counts_t, counts_tp, emb, w1t, b1, w2t, b2, cbt, cb):
    nblk = _A // _BB
    return pl.pallas_call(
        _tc_body,
        grid=(nblk,),
        in_specs=[
            pl.BlockSpec((_BB, _K), lambda i: (i, 0)),
            pl.BlockSpec((_BB, _K), lambda i: (i, 0)),
            pl.BlockSpec((_K, _DIM), lambda i: (0, 0)),
            pl.BlockSpec((2 * _DIM, _DIM), lambda i: (0, 0)),
            pl.BlockSpec((1, _DIM), lambda i: (0, 0)),
            pl.BlockSpec((_DIM, _DIM), lambda i: (0, 0)),
            pl.BlockSpec((1, _DIM), lambda i: (0, 0)),
            pl.BlockSpec((_DIM, _A), lambda i: (0, 0)),
            pl.BlockSpec((_A, _DIM), lambda i: (0, 0)),
        ],
        out_specs=[
            pl.BlockSpec((_BB, 1), lambda i: (i, 0)),
            pl.BlockSpec((_BB, _DIM), lambda i: (i, 0)),
        ],
        out_shape=[
            jax.ShapeDtypeStruct((_A, 1), jnp.int32),
            jax.ShapeDtypeStruct((_A, _DIM), jnp.float32),
        ],
    )(counts_t, counts_tp, emb, w1t, b1, w2t, b2, cbt, cb)


def kernel(z_t, z_tp1, embed_w, W1, b1, W2, b2, codebook):
    B = z_t.shape[0]
    counts_t, counts_tp = _get_sc_counts()(
        z_t.reshape(B, -1).astype(jnp.int32),
        z_tp1.reshape(B, -1).astype(jnp.int32))
    idx2, q = _tc_call(counts_t, counts_tp, embed_w, W1.T, b1.reshape(1, -1),
                       W2.T, b2.reshape(1, -1), codebook.T, codebook)
    return (idx2.reshape(B), q)
